# Initial kernel scaffold; baseline (speedup 1.0000x reference)
import jax
import jax.numpy as jnp
from jax.experimental import pallas as pl


def kernel(values, offsets):
    raise NotImplementedError("write your pallas kernel here")


# speedup vs baseline: 9.6028x; 9.6028x over previous
import functools

import jax
import jax.numpy as jnp
from jax import lax
from jax.experimental import pallas as pl
from jax.experimental.pallas import tpu as pltpu
from jax.experimental.pallas import tpu_sc as plsc

_B = 16
_PAD = 4096
_HALF = _PAD // 2
_VEC = 16
_NV = _HALF // _VEC
_INBUF = _HALF + _VEC


def _body(values_hbm, offs_hbm, out_hbm, off_v, in_v, out_v):
    c = lax.axis_index("c")
    s = lax.axis_index("s")
    _tile(c, s, values_hbm, offs_hbm, out_hbm, off_v, in_v, out_v)


def _tile(c, s, values_hbm, offs_hbm, out_hbm, off_v, in_v, out_v):
    pltpu.sync_copy(offs_hbm, off_v)

    start = off_v[pl.ds(s, _VEC)][0]
    end = off_v[pl.ds(s + 1, _VEC)][0]

    base = (start >> 3) << 3
    r = start - base
    src = pl.multiple_of(base + c * _HALF, 8)
    pltpu.sync_copy(values_hbm.at[pl.ds(src, _INBUF)], in_v)

    limit = (end - start) - c * _HALF

    def step(j, _):
        o = j * _VEC
        v = in_v[pl.ds(r + o, _VEC)]
        col = o + lax.iota(jnp.int32, _VEC)
        out_v[pl.ds(o, _VEC)] = jnp.where(col < limit, v, 0.0)
        return _

    lax.fori_loop(0, _NV, step, None)
    pltpu.sync_copy(out_v, out_hbm.at[s * 2 + c])


def kernel(values, offsets):
    values_p = jnp.pad(values, (0, _PAD + 2 * _VEC))
    offs_p = jnp.pad(offsets.astype(jnp.int32), (0, 32 - offsets.shape[0]))

    mesh = plsc.VectorSubcoreMesh(
        core_axis_name="c", subcore_axis_name="s", num_cores=2, num_subcores=16)
    run = pl.kernel(
        _body,
        out_type=jax.ShapeDtypeStruct((2 * _B, _HALF), jnp.float32),
        mesh=mesh,
        scratch_types=[
            pltpu.VMEM((32,), jnp.int32),
            pltpu.VMEM((_INBUF,), jnp.float32),
            pltpu.VMEM((_HALF,), jnp.float32),
        ],
    )
    out32 = run(values_p, offs_p)
    return out32.reshape(_B, _PAD)
